# baseline (speedup 1.0000x reference)
import jax
import jax.numpy as jnp
from jax.experimental import pallas as pl
from jax.experimental.pallas import tpu as pltpu

_NBUF = 6
_CHUNK = 1 << 20


def _chunks(total):
    out, off = [], 0
    while off < total:
        sz = min(_CHUNK, total - off)
        out.append((off, sz))
        off += sz
    return out


def _ring_copy(pairs, ibufs, obufs, isems, osems):
    def in_cp(k, j):
        src, dst, off, sz = pairs[k]
        return pltpu.make_async_copy(
            src.at[pl.ds(off, sz)], ibufs[j].at[pl.ds(0, sz)], isems.at[j])

    def out_cp(k, j):
        src, dst, off, sz = pairs[k]
        return pltpu.make_async_copy(
            obufs[j].at[pl.ds(0, sz)], dst.at[pl.ds(off, sz)], osems.at[j])

    n = len(pairs)
    for k in range(min(_NBUF, n)):
        in_cp(k, k).start()
    for k in range(n):
        j = k % _NBUF
        in_cp(k, j).wait()
        if k >= _NBUF:
            out_cp(k - _NBUF, j).wait()
        sz = pairs[k][3]
        obufs[j][pl.ds(0, sz)] = ibufs[j][pl.ds(0, sz)]
        out_cp(k, j).start()
        if k + _NBUF < n:
            in_cp(k + _NBUF, j).start()
    for k in range(max(0, n - _NBUF), n):
        out_cp(k, k % _NBUF).wait()


def _body(u_in, i_in, u_out, i_out, *scratch):
    ibufs = scratch[:_NBUF]
    obufs = scratch[_NBUF:2 * _NBUF]
    isems, osems = scratch[2 * _NBUF], scratch[2 * _NBUF + 1]
    pairs = [(u_in, u_out, off, sz) for off, sz in _chunks(u_in.shape[0])]
    pairs += [(i_in, i_out, off, sz) for off, sz in _chunks(i_in.shape[0])]
    _ring_copy(pairs, ibufs, obufs, isems, osems)


def kernel(embed_user, embed_item):
    u_shape, i_shape = embed_user.shape, embed_item.shape
    u_flat = embed_user.T.reshape(-1)
    i_flat = embed_item.T.reshape(-1)
    out_u, out_i = pl.pallas_call(
        _body,
        in_specs=[
            pl.BlockSpec(memory_space=pltpu.HBM),
            pl.BlockSpec(memory_space=pltpu.HBM),
        ],
        out_specs=[
            pl.BlockSpec(memory_space=pltpu.HBM),
            pl.BlockSpec(memory_space=pltpu.HBM),
        ],
        out_shape=[
            jax.ShapeDtypeStruct(u_flat.shape, u_flat.dtype),
            jax.ShapeDtypeStruct(i_flat.shape, i_flat.dtype),
        ],
        scratch_shapes=(
            [pltpu.VMEM((_CHUNK,), jnp.float32) for _ in range(2 * _NBUF)]
            + [pltpu.SemaphoreType.DMA((_NBUF,)),
               pltpu.SemaphoreType.DMA((_NBUF,))]
        ),
    )(u_flat, i_flat)
    out_u = out_u.reshape(u_shape[1], u_shape[0]).T
    out_i = out_i.reshape(i_shape[1], i_shape[0]).T
    return (out_u, out_i)


# speedup vs baseline: 62.5263x; 62.5263x over previous
import jax
import jax.numpy as jnp
from jax.experimental import pallas as pl
from jax.experimental.pallas import tpu as pltpu

_BLOCK_COLS = 8192


def _make_body(nblk_i):
    def _copy_body(u_ref, i_ref, uo_ref, io_ref):
        uo_ref[...] = u_ref[...]

        @pl.when(pl.program_id(0) < nblk_i)
        def _():
            io_ref[...] = i_ref[...]

    return _copy_body


def kernel(embed_user, embed_item):
    ut = embed_user.T
    it = embed_item.T
    dim, ucols = ut.shape
    icols = it.shape[1]
    nblk_u = (ucols + _BLOCK_COLS - 1) // _BLOCK_COLS
    nblk_i = (icols + _BLOCK_COLS - 1) // _BLOCK_COLS

    u_spec = pl.BlockSpec((dim, _BLOCK_COLS), lambda j: (0, j))
    i_spec = pl.BlockSpec((dim, _BLOCK_COLS),
                          lambda j: (0, jnp.minimum(j, nblk_i - 1)))
    out_u, out_i = pl.pallas_call(
        _make_body(nblk_i),
        grid=(max(nblk_u, nblk_i),),
        in_specs=[u_spec, i_spec],
        out_specs=[u_spec, i_spec],
        out_shape=[
            jax.ShapeDtypeStruct(ut.shape, ut.dtype),
            jax.ShapeDtypeStruct(it.shape, it.dtype),
        ],
        compiler_params=pltpu.CompilerParams(
            dimension_semantics=("arbitrary",),
        ),
    )(ut, it)
    return (out_u.T, out_i.T)


# speedup vs baseline: 67.6591x; 1.0821x over previous
import jax
import jax.numpy as jnp
from jax.experimental import pallas as pl
from jax.experimental.pallas import tpu as pltpu

_BLOCK_COLS = 16384


def _make_body(nblk_i):
    def _copy_body(u_ref, i_ref, uo_ref, io_ref):
        uo_ref[...] = u_ref[...]

        @pl.when(pl.program_id(0) < nblk_i)
        def _():
            io_ref[...] = i_ref[...]

    return _copy_body


def kernel(embed_user, embed_item):
    ut = embed_user.T
    it = embed_item.T
    dim, ucols = ut.shape
    icols = it.shape[1]
    nblk_u = (ucols + _BLOCK_COLS - 1) // _BLOCK_COLS
    nblk_i = (icols + _BLOCK_COLS - 1) // _BLOCK_COLS

    u_spec = pl.BlockSpec((dim, _BLOCK_COLS), lambda j: (0, j))
    i_spec = pl.BlockSpec((dim, _BLOCK_COLS),
                          lambda j: (0, jnp.minimum(j, nblk_i - 1)))
    out_u, out_i = pl.pallas_call(
        _make_body(nblk_i),
        grid=(max(nblk_u, nblk_i),),
        in_specs=[u_spec, i_spec],
        out_specs=[u_spec, i_spec],
        out_shape=[
            jax.ShapeDtypeStruct(ut.shape, ut.dtype),
            jax.ShapeDtypeStruct(it.shape, it.dtype),
        ],
        compiler_params=pltpu.CompilerParams(
            dimension_semantics=("arbitrary",),
        ),
    )(ut, it)
    return (out_u.T, out_i.T)


# speedup vs baseline: 68.8114x; 1.0170x over previous
import jax
import jax.numpy as jnp
from jax.experimental import pallas as pl
from jax.experimental.pallas import tpu as pltpu

_BLOCK_COLS = 24576


def _make_body(nblk_i):
    def _copy_body(u_ref, i_ref, uo_ref, io_ref):
        uo_ref[...] = u_ref[...]

        @pl.when(pl.program_id(0) < nblk_i)
        def _():
            io_ref[...] = i_ref[...]

    return _copy_body


def kernel(embed_user, embed_item):
    ut = embed_user.T
    it = embed_item.T
    dim, ucols = ut.shape
    icols = it.shape[1]
    nblk_u = (ucols + _BLOCK_COLS - 1) // _BLOCK_COLS
    nblk_i = (icols + _BLOCK_COLS - 1) // _BLOCK_COLS

    u_spec = pl.BlockSpec((dim, _BLOCK_COLS), lambda j: (0, j))
    i_spec = pl.BlockSpec((dim, _BLOCK_COLS),
                          lambda j: (0, jnp.minimum(j, nblk_i - 1)))
    out_u, out_i = pl.pallas_call(
        _make_body(nblk_i),
        grid=(max(nblk_u, nblk_i),),
        in_specs=[u_spec, i_spec],
        out_specs=[u_spec, i_spec],
        out_shape=[
            jax.ShapeDtypeStruct(ut.shape, ut.dtype),
            jax.ShapeDtypeStruct(it.shape, it.dtype),
        ],
        compiler_params=pltpu.CompilerParams(
            dimension_semantics=("arbitrary",),
        ),
    )(ut, it)
    return (out_u.T, out_i.T)


# speedup vs baseline: 68.9581x; 1.0021x over previous
import jax
import jax.numpy as jnp
from jax.experimental import pallas as pl
from jax.experimental.pallas import tpu as pltpu

_BLOCK_COLS = 32768
_BLOCK_COLS_I = 8192


def _make_body(nblk_i):
    def _copy_body(u_ref, i_ref, uo_ref, io_ref):
        uo_ref[...] = u_ref[...]

        @pl.when(pl.program_id(0) < nblk_i)
        def _():
            io_ref[...] = i_ref[...]

    return _copy_body


def kernel(embed_user, embed_item):
    ut = embed_user.T
    it = embed_item.T
    dim, ucols = ut.shape
    icols = it.shape[1]
    nblk_u = (ucols + _BLOCK_COLS - 1) // _BLOCK_COLS
    nblk_i = (icols + _BLOCK_COLS_I - 1) // _BLOCK_COLS_I

    u_spec = pl.BlockSpec((dim, _BLOCK_COLS), lambda j: (0, j))
    i_spec = pl.BlockSpec((dim, _BLOCK_COLS_I),
                          lambda j: (0, jnp.minimum(j, nblk_i - 1)))
    out_u, out_i = pl.pallas_call(
        _make_body(nblk_i),
        grid=(max(nblk_u, nblk_i),),
        in_specs=[u_spec, i_spec],
        out_specs=[u_spec, i_spec],
        out_shape=[
            jax.ShapeDtypeStruct(ut.shape, ut.dtype),
            jax.ShapeDtypeStruct(it.shape, it.dtype),
        ],
        compiler_params=pltpu.CompilerParams(
            dimension_semantics=("arbitrary",),
        ),
    )(ut, it)
    return (out_u.T, out_i.T)


# speedup vs baseline: 69.1252x; 1.0024x over previous
import jax
import jax.numpy as jnp
from jax.experimental import pallas as pl
from jax.experimental.pallas import tpu as pltpu

_BLOCK_COLS = 49152
_BLOCK_COLS_I = 8192


def _make_body(nblk_i):
    def _copy_body(u_ref, i_ref, uo_ref, io_ref):
        uo_ref[...] = u_ref[...]

        @pl.when(pl.program_id(0) < nblk_i)
        def _():
            io_ref[...] = i_ref[...]

    return _copy_body


def kernel(embed_user, embed_item):
    ut = embed_user.T
    it = embed_item.T
    dim, ucols = ut.shape
    icols = it.shape[1]
    nblk_u = (ucols + _BLOCK_COLS - 1) // _BLOCK_COLS
    nblk_i = (icols + _BLOCK_COLS_I - 1) // _BLOCK_COLS_I

    u_spec = pl.BlockSpec((dim, _BLOCK_COLS), lambda j: (0, j))
    i_spec = pl.BlockSpec((dim, _BLOCK_COLS_I),
                          lambda j: (0, jnp.minimum(j, nblk_i - 1)))
    out_u, out_i = pl.pallas_call(
        _make_body(nblk_i),
        grid=(max(nblk_u, nblk_i),),
        in_specs=[u_spec, i_spec],
        out_specs=[u_spec, i_spec],
        out_shape=[
            jax.ShapeDtypeStruct(ut.shape, ut.dtype),
            jax.ShapeDtypeStruct(it.shape, it.dtype),
        ],
        compiler_params=pltpu.CompilerParams(
            dimension_semantics=("arbitrary",),
        ),
    )(ut, it)
    return (out_u.T, out_i.T)
